# Initial kernel scaffold; baseline (speedup 1.0000x reference)
import jax
import jax.numpy as jnp
from jax.experimental import pallas as pl


def kernel(x, edge_index, W1, b1, W2, b2, W3, b3):
    raise NotImplementedError("write your pallas kernel here")


# speedup vs baseline: 5.6444x; 5.6444x over previous
import functools

import jax
import jax.numpy as jnp
from jax import lax
from jax.experimental import pallas as pl
from jax.experimental.pallas import tpu as pltpu
from jax.experimental.pallas import tpu_sc as plsc

N = 10000
E = 320000
D = 128
NC = 2
NS = 16
NW = NC * NS
CH = 40
NCH = E // (CH * NW)
RPS = N // NS
DW = 16
ZR = 125
NBLK = 10
BR = N // NBLK


def _mesh():
    return plsc.VectorSubcoreMesh(
        core_axis_name="c", subcore_axis_name="s",
        num_cores=NC, num_subcores=NS)


@functools.partial(
    pl.kernel,
    out_type=(jax.ShapeDtypeStruct((NC, N, DW), jnp.float32),
              jax.ShapeDtypeStruct((NC, N, DW), jnp.float32)),
    mesh=_mesh(),
    compiler_params=pltpu.CompilerParams(use_tc_tiling_on_sc=False),
    scratch_types=[
        pltpu.VMEM((NCH, CH), jnp.int32),
        pltpu.VMEM((NCH, CH), jnp.int32),
        pltpu.VMEM((CH, DW), jnp.float32),
        pltpu.VMEM((RPS, DW), jnp.float32),
        pltpu.VMEM_SHARED((N, DW), jnp.float32),
        pltpu.VMEM_SHARED((N, DW), jnp.float32),
    ],
)
def _sc_degrees(src_hbm, dst_hbm, od_hbm, id_hbm,
                src_v, dst_v, ones_v, zbuf, od_sh, id_sh):
    c = lax.axis_index("c")
    s = lax.axis_index("s")
    wid = c * NS + s

    def fill_z(i, _):
        zbuf[i, :] = jnp.zeros((16,), jnp.float32)
        return 0
    lax.fori_loop(0, RPS, fill_z, 0)

    def fill_o(i, _):
        ones_v[i, :] = jnp.ones((16,), jnp.float32)
        return 0
    lax.fori_loop(0, CH, fill_o, 0)

    rb = s * RPS
    pltpu.sync_copy(zbuf, od_sh.at[pl.ds(rb, RPS)])
    pltpu.sync_copy(zbuf, id_sh.at[pl.ds(rb, RPS)])
    pltpu.sync_copy(src_hbm.at[wid], src_v)
    pltpu.sync_copy(dst_hbm.at[wid], dst_v)
    plsc.subcore_barrier()

    def body(j, _):
        pltpu.sync_copy(ones_v, od_sh.at[src_v.at[j]], add=True)
        pltpu.sync_copy(ones_v, id_sh.at[dst_v.at[j]], add=True)
        return 0
    lax.fori_loop(0, NCH, body, 0)
    plsc.subcore_barrier()

    @pl.when(s == 0)
    def _():
        pltpu.sync_copy(od_sh, od_hbm.at[c])
        pltpu.sync_copy(id_sh, id_hbm.at[c])


@functools.partial(
    pl.kernel,
    out_type=jax.ShapeDtypeStruct((NC, N, D), jnp.float32),
    mesh=_mesh(),
    compiler_params=pltpu.CompilerParams(use_tc_tiling_on_sc=False),
    scratch_types=[
        pltpu.VMEM((NCH, CH), jnp.int32),
        pltpu.VMEM((NCH, CH), jnp.int32),
        pltpu.VMEM((CH, D), jnp.float32),
        pltpu.VMEM((ZR, D), jnp.float32),
        pltpu.VMEM_SHARED((N, D), jnp.float32),
        pltpu.SemaphoreType.DMA,
    ],
)
def _sc_edge_agg(y_hbm, src_hbm, dst_hbm, out_hbm,
                 src_v, dst_v, rowbuf, zbuf, agg_sh, sem):
    c = lax.axis_index("c")
    s = lax.axis_index("s")
    wid = c * NS + s

    def fill_z(i, _):
        for k in range(D // 16):
            zbuf[i, pl.ds(k * 16, 16)] = jnp.zeros((16,), jnp.float32)
        return 0
    lax.fori_loop(0, ZR, fill_z, 0)

    rb = s * RPS
    for k in range(RPS // ZR):
        pltpu.sync_copy(zbuf, agg_sh.at[pl.ds(rb + k * ZR, ZR)])
    pltpu.sync_copy(src_hbm.at[wid], src_v)
    pltpu.sync_copy(dst_hbm.at[wid], dst_v)
    plsc.subcore_barrier()

    def body(j, _):
        pltpu.async_copy(y_hbm.at[src_v.at[j]], rowbuf, sem).wait()
        pltpu.sync_copy(rowbuf, agg_sh.at[dst_v.at[j]], add=True)
        return 0
    lax.fori_loop(0, NCH, body, 0)
    plsc.subcore_barrier()

    @pl.when(s == 0)
    def _():
        pltpu.sync_copy(agg_sh, out_hbm.at[c])


def _tc1_body(x_ref, od_ref, id_ref, w_ref, y_ref, ns_ref, nd_ref):
    od = od_ref[0, :, 0:1] + od_ref[1, :, 0:1]
    ind = id_ref[0, :, 0:1] + id_ref[1, :, 0:1]
    ns = lax.rsqrt(jnp.maximum(od, 1.0))
    nd = lax.rsqrt(jnp.maximum(ind, 1.0))
    y_ref[...] = jnp.dot(x_ref[...] * ns, w_ref[...],
                         preferred_element_type=jnp.float32)
    ns_ref[...] = ns
    nd_ref[...] = nd


_tc1 = pl.pallas_call(
    _tc1_body,
    grid=(NBLK,),
    in_specs=[
        pl.BlockSpec((BR, D), lambda i: (i, 0)),
        pl.BlockSpec((NC, BR, DW), lambda i: (0, i, 0)),
        pl.BlockSpec((NC, BR, DW), lambda i: (0, i, 0)),
        pl.BlockSpec((D, D), lambda i: (0, 0)),
    ],
    out_specs=[
        pl.BlockSpec((BR, D), lambda i: (i, 0)),
        pl.BlockSpec((BR, 1), lambda i: (i, 0)),
        pl.BlockSpec((BR, 1), lambda i: (i, 0)),
    ],
    out_shape=[
        jax.ShapeDtypeStruct((N, D), jnp.float32),
        jax.ShapeDtypeStruct((N, 1), jnp.float32),
        jax.ShapeDtypeStruct((N, 1), jnp.float32),
    ],
)


def _tc2_body(a_ref, nd_ref, ns_ref, b_ref, w_ref, y_ref):
    a = a_ref[0] + a_ref[1]
    h = jnp.maximum(a * nd_ref[...] + b_ref[...], 0.0)
    y_ref[...] = jnp.dot(h * ns_ref[...], w_ref[...],
                         preferred_element_type=jnp.float32)


_tc2 = pl.pallas_call(
    _tc2_body,
    grid=(NBLK,),
    in_specs=[
        pl.BlockSpec((NC, BR, D), lambda i: (0, i, 0)),
        pl.BlockSpec((BR, 1), lambda i: (i, 0)),
        pl.BlockSpec((BR, 1), lambda i: (i, 0)),
        pl.BlockSpec((1, D), lambda i: (0, 0)),
        pl.BlockSpec((D, D), lambda i: (0, 0)),
    ],
    out_specs=pl.BlockSpec((BR, D), lambda i: (i, 0)),
    out_shape=jax.ShapeDtypeStruct((N, D), jnp.float32),
)


def _tc3_body(a_ref, nd_ref, b_ref, w3_ref, b3_ref, o_ref, acc_ref):
    i = pl.program_id(0)
    a = a_ref[0] + a_ref[1]
    h = jnp.maximum(a * nd_ref[...] + b_ref[...], 0.0)
    p = jnp.sum(h, axis=0, keepdims=True)

    @pl.when(i == 0)
    def _():
        acc_ref[...] = jnp.zeros_like(acc_ref)

    acc_ref[...] += p

    @pl.when(i == NBLK - 1)
    def _():
        o_ref[...] = (jnp.sum(acc_ref[...] * (1.0 / N) * w3_ref[...],
                              axis=1, keepdims=True) + b3_ref[...])


_tc3 = pl.pallas_call(
    _tc3_body,
    grid=(NBLK,),
    in_specs=[
        pl.BlockSpec((NC, BR, D), lambda i: (0, i, 0)),
        pl.BlockSpec((BR, 1), lambda i: (i, 0)),
        pl.BlockSpec((1, D), lambda i: (0, 0)),
        pl.BlockSpec((1, D), lambda i: (0, 0)),
        pl.BlockSpec((1, 1), lambda i: (0, 0)),
    ],
    out_specs=pl.BlockSpec((1, 1), lambda i: (0, 0)),
    out_shape=jax.ShapeDtypeStruct((1, 1), jnp.float32),
    scratch_shapes=[pltpu.VMEM((1, D), jnp.float32)],
)


def kernel(x, edge_index, W1, b1, W2, b2, W3, b3):
    src3 = edge_index[0].reshape(NW, NCH, CH)
    dst3 = edge_index[1].reshape(NW, NCH, CH)
    od, ind = _sc_degrees(src3, dst3)
    y1, ns, nd = _tc1(x, od, ind, W1)
    agg1 = _sc_edge_agg(y1, src3, dst3)
    y2 = _tc2(agg1, nd, ns, b1.reshape(1, D), W2)
    agg2 = _sc_edge_agg(y2, src3, dst3)
    out = _tc3(agg2, nd, b2.reshape(1, D), W3.reshape(1, D),
               b3.reshape(1, 1))
    return out
